# baseline (speedup 1.0000x reference)
import jax
import jax.numpy as jnp
from jax import lax
from jax.experimental import pallas as pl
from jax.experimental.pallas import tpu as pltpu
from jax.experimental.pallas import tpu_sc as plsc

T_TEXT = 200
EMBED_DIM = 1024
CHUNK0 = 104
CHUNK1 = 96
RING = 16


def _do_rows(idx_hbm, table_hbm, out_hbm, idx_s, sem, base, count):
    pltpu.sync_copy(idx_hbm, idx_s)

    def issue(i):
        row = idx_s[base + i]
        return pltpu.make_async_copy(
            table_hbm.at[row], out_hbm.at[base + i], sem
        )

    def body(i, _):
        issue(i).start()

        @pl.when(i >= RING)
        def _():
            issue(i - RING).wait()

        return 0

    lax.fori_loop(0, count, body, 0)

    def drain(i, _):
        issue(i).wait()
        return 0

    lax.fori_loop(count - RING, count, drain, 0)


def _gather_body(idx_hbm, table_hbm, out_hbm, idx_s, sem):
    cid = lax.axis_index("c")

    @pl.when(cid == 0)
    def _():
        _do_rows(idx_hbm, table_hbm, out_hbm, idx_s, sem, 0, CHUNK0)

    @pl.when(cid == 1)
    def _():
        _do_rows(idx_hbm, table_hbm, out_hbm, idx_s, sem, CHUNK0, CHUNK1)


def kernel(token_ids, embed_table):
    idx = token_ids.reshape(T_TEXT).astype(jnp.int32)
    idx_pad = jnp.zeros((256,), jnp.int32).at[:T_TEXT].set(idx)
    mesh = plsc.ScalarSubcoreMesh(axis_name="c", num_cores=2)
    out = pl.kernel(
        _gather_body,
        mesh=mesh,
        out_type=jax.ShapeDtypeStruct((T_TEXT, EMBED_DIM), jnp.float32),
        scratch_types=[
            pltpu.SMEM((256,), jnp.int32),
            pltpu.SemaphoreType.DMA,
        ],
    )(idx_pad, embed_table)
    return out.reshape(1, T_TEXT, EMBED_DIM)


# speedup vs baseline: 2.1174x; 2.1174x over previous
import jax
import jax.numpy as jnp
from jax import lax
from jax.experimental import pallas as pl
from jax.experimental.pallas import tpu as pltpu
from jax.experimental.pallas import tpu_sc as plsc

T_TEXT = 200
EMBED_DIM = 1024
ROWS_MAIN = 16
HALF = ROWS_MAIN // 2
NUM_MAIN = 12
ROWS_TAIL = 8


def _gather_body(idx_hbm, table_hbm, out_hbm, idx_v, rows_v, sem_a, sem_b, sem_w):
    wid = lax.axis_index("s")

    @pl.when(wid < NUM_MAIN)
    def _():
        base = wid * ROWS_MAIN
        pltpu.sync_copy(idx_hbm.at[pl.ds(base, ROWS_MAIN)], idx_v)
        ga = pltpu.make_async_copy(
            table_hbm.at[idx_v.at[pl.ds(0, HALF)]], rows_v.at[pl.ds(0, HALF)],
            sem_a,
        )
        gb = pltpu.make_async_copy(
            table_hbm.at[idx_v.at[pl.ds(HALF, HALF)]],
            rows_v.at[pl.ds(HALF, HALF)], sem_b,
        )
        ga.start()
        gb.start()
        ga.wait()
        wa = pltpu.make_async_copy(
            rows_v.at[pl.ds(0, HALF)], out_hbm.at[pl.ds(base, HALF)], sem_w
        )
        wa.start()
        gb.wait()
        wb = pltpu.make_async_copy(
            rows_v.at[pl.ds(HALF, HALF)], out_hbm.at[pl.ds(base + HALF, HALF)],
            sem_w,
        )
        wb.start()
        wa.wait()
        wb.wait()

    @pl.when(wid == NUM_MAIN)
    def _():
        base = NUM_MAIN * ROWS_MAIN
        pltpu.sync_copy(
            idx_hbm.at[pl.ds(base, ROWS_TAIL)], idx_v.at[pl.ds(0, ROWS_TAIL)]
        )
        pltpu.async_copy(
            table_hbm.at[idx_v.at[pl.ds(0, ROWS_TAIL)]],
            rows_v.at[pl.ds(0, ROWS_TAIL)],
            sem_a,
        ).wait()
        pltpu.sync_copy(
            rows_v.at[pl.ds(0, ROWS_TAIL)], out_hbm.at[pl.ds(base, ROWS_TAIL)]
        )


def kernel(token_ids, embed_table):
    idx = token_ids.reshape(T_TEXT).astype(jnp.int32)
    mesh = plsc.VectorSubcoreMesh(
        core_axis_name="c", subcore_axis_name="s", num_cores=1
    )
    out = pl.kernel(
        _gather_body,
        mesh=mesh,
        out_type=jax.ShapeDtypeStruct((T_TEXT, EMBED_DIM), jnp.float32),
        scratch_types=[
            pltpu.VMEM((ROWS_MAIN,), jnp.int32),
            pltpu.VMEM((ROWS_MAIN, EMBED_DIM), jnp.float32),
            pltpu.SemaphoreType.DMA,
            pltpu.SemaphoreType.DMA,
            pltpu.SemaphoreType.DMA,
        ],
    )(idx, embed_table)
    return out.reshape(1, T_TEXT, EMBED_DIM)
